# baseline (speedup 1.0000x reference)
import jax
import jax.numpy as jnp
from jax.experimental import pallas as pl
from jax.experimental.pallas import tpu as pltpu

C = 32
F = 256
H = 32
HH = H * H
P = C * C
CHUNK = 128
NCHUNK = HH // CHUNK
BPS = 2


def _body(xT_ref, WinT_ref, bin_ref, WoutT_ref, bout_ref,
          Wcat_0_ref, WsT_0_ref, bsT_0_ref,
          Wcat_1_ref, WsT_1_ref, bsT_1_ref,
          yT_ref):
    bf16 = jnp.bfloat16
    iota_r = jax.lax.broadcasted_iota(jnp.int32, (C, P), 0)
    iota_p = jax.lax.broadcasted_iota(jnp.int32, (C, P), 1)
    R = (iota_p // C == iota_r).astype(bf16)
    Rd = (iota_p % C == iota_r).astype(bf16)
    iota_pp = jax.lax.broadcasted_iota(jnp.int32, (P, C), 0)
    iota_d = jax.lax.broadcasted_iota(jnp.int32, (P, C), 1)
    S = ((iota_pp % C == iota_d) &
         (iota_pp // C != iota_d)).astype(bf16)
    iota_j = jax.lax.broadcasted_iota(jnp.int32, (CHUNK // H, CHUNK), 0)
    iota_k = jax.lax.broadcasted_iota(jnp.int32, (CHUNK // H, CHUNK), 1)
    S2 = (iota_k // H == iota_j).astype(bf16)

    inv_deg = 1.0 / float(C - 1)

    for bb in range(BPS):
        xT = xT_ref[bb]

        hT = jnp.maximum(
            jnp.dot(WinT_ref[...], xT, preferred_element_type=jnp.float32)
            + bin_ref[...], 0.0)

        _one_graph(bb, xT, hT, R, Rd, S, S2, inv_deg,
                   WoutT_ref, bout_ref,
                   ((Wcat_0_ref, WsT_0_ref, bsT_0_ref),
                    (Wcat_1_ref, WsT_1_ref, bsT_1_ref)),
                   yT_ref)


def _one_graph(bb, xT, hT, R, Rd, S, S2, inv_deg, WoutT_ref, bout_ref,
               layers, yT_ref):
    bf16 = jnp.bfloat16
    hb0 = hT.astype(bf16)
    hsrcT = jnp.dot(hb0, R, preferred_element_type=jnp.float32)
    hdstT = jnp.dot(hb0, Rd, preferred_element_type=jnp.float32)
    numf = jnp.sum(hsrcT * hdstT, axis=0, keepdims=True)
    nsrc = jnp.maximum(
        jnp.sqrt(jnp.sum(hsrcT * hsrcT, axis=0, keepdims=True)), 1e-8)
    ndst = jnp.maximum(
        jnp.sqrt(jnp.sum(hdstT * hdstT, axis=0, keepdims=True)), 1e-8)
    cosf = numf / (nsrc * ndst)
    dvec = hdstT - hsrcT
    distr = jnp.sqrt(jnp.sum(dvec * dvec, axis=0, keepdims=True))
    dmean = jnp.sum(distr) / float(C * (C - 1))
    distf = distr / (dmean + 1e-6)
    attr3 = jnp.concatenate(
        [cosf, distf, jnp.ones((1, P), jnp.float32)], axis=0).astype(bf16)

    for Wcat, WsT, bsT in layers:
        hrepT = jnp.dot(hT.astype(bf16), R,
                        preferred_element_type=jnp.float32).astype(bf16)
        hrep_c = jnp.concatenate([hrepT] * (CHUNK // H), axis=0)
        msg_parts = []
        for c in range(NCHUNK):
            r0 = c * CHUNK
            pre = jnp.dot(Wcat[r0:r0 + CHUNK, :], attr3,
                          preferred_element_type=jnp.float32)
            M = jnp.tanh(pre).astype(bf16) * hrep_c
            msg_parts.append(
                jnp.dot(S2, M, preferred_element_type=jnp.float32))
        MSG = jnp.concatenate(msg_parts, axis=0)
        AGG = jnp.dot(MSG.astype(bf16), S,
                      preferred_element_type=jnp.float32)
        hT = jnp.maximum(
            AGG * inv_deg
            + jnp.dot(WsT[...], hT, preferred_element_type=jnp.float32)
            + bsT[...], 0.0)

    yT_ref[bb] = xT + jnp.dot(WoutT_ref[...], hT,
                              preferred_element_type=jnp.float32) + bout_ref[...]


def kernel(x, W_in, b_in, W_out, b_out,
           We_0, be_0, Ws_0, bs_0, We_1, be_1, Ws_1, bs_1):
    B = x.shape[0]
    f32 = jnp.float32

    xT = x.transpose(0, 2, 1)
    WinT = W_in.T
    WoutT = W_out.T
    binT = b_in[:, None]
    boutT = b_out[:, None]

    def edge_cat(We, be):
        Wp = We.reshape(2, H, H).transpose(0, 2, 1).reshape(2, HH)
        bp = be.reshape(H, H).T.reshape(HH)
        return jnp.stack([Wp[0], Wp[1], bp], axis=1).astype(jnp.bfloat16)

    Wcat_0 = edge_cat(We_0, be_0)
    Wcat_1 = edge_cat(We_1, be_1)
    WsT_0, bsT_0 = Ws_0.T, bs_0[:, None]
    WsT_1, bsT_1 = Ws_1.T, bs_1[:, None]

    full = lambda shape: pl.BlockSpec(shape, lambda b: (0,) * len(shape))
    grid_spec = pl.GridSpec(
        grid=(B // BPS,),
        in_specs=[
            pl.BlockSpec((BPS, F, C), lambda b: (b, 0, 0)),
            full((H, F)), full((H, 1)), full((F, H)), full((F, 1)),
            full((HH, 3)), full((H, H)), full((H, 1)),
            full((HH, 3)), full((H, H)), full((H, 1)),
        ],
        out_specs=pl.BlockSpec((BPS, F, C), lambda b: (b, 0, 0)),
    )
    yT = pl.pallas_call(
        _body,
        grid_spec=grid_spec,
        out_shape=jax.ShapeDtypeStruct((B, F, C), f32),
        compiler_params=pltpu.CompilerParams(
            dimension_semantics=("parallel",)),
    )(xT.astype(f32), WinT, binT, WoutT, boutT,
      Wcat_0, WsT_0, bsT_0, Wcat_1, WsT_1, bsT_1)
    return yT.transpose(0, 2, 1)


# speedup vs baseline: 1.4332x; 1.4332x over previous
import numpy as np
import jax
import jax.numpy as jnp
from jax.experimental import pallas as pl
from jax.experimental.pallas import tpu as pltpu

C = 32
F = 256
H = 32
HH = H * H
P = C * C
E = C * (C - 1)
BPS = 2
PL = BPS * P
CL = BPS * C
CHUNK = 128
NCHUNK = HH // CHUNK


def _body(xT_ref, WinT_ref, bin_ref, WoutT_ref, bout_ref,
          Rbig_ref, Rdbig_ref, Sbig_ref, Bsum_ref, Bbc_ref, S2_ref,
          Wcat_0_ref, WsT_0_ref, bsT_0_ref,
          Wcat_1_ref, WsT_1_ref, bsT_1_ref,
          yT_ref):
    bf16 = jnp.bfloat16
    xT = xT_ref[0]

    hT = jnp.maximum(
        jnp.dot(WinT_ref[...], xT, preferred_element_type=jnp.float32)
        + bin_ref[...], 0.0)

    Rbig = Rbig_ref[...]
    hb0 = hT.astype(bf16)
    hsrcT = jnp.dot(hb0, Rbig, preferred_element_type=jnp.float32)
    hdstT = jnp.dot(hb0, Rdbig_ref[...], preferred_element_type=jnp.float32)
    numf = jnp.sum(hsrcT * hdstT, axis=0, keepdims=True)
    nsrc = jnp.maximum(
        jnp.sqrt(jnp.sum(hsrcT * hsrcT, axis=0, keepdims=True)), 1e-8)
    ndst = jnp.maximum(
        jnp.sqrt(jnp.sum(hdstT * hdstT, axis=0, keepdims=True)), 1e-8)
    cosf = numf / (nsrc * ndst)
    dvec = hdstT - hsrcT
    distr = jnp.sqrt(jnp.sum(dvec * dvec, axis=0, keepdims=True))
    bsums = jnp.dot(distr.astype(bf16), Bsum_ref[...],
                    preferred_element_type=jnp.float32)
    dmean = jnp.dot(bsums.astype(bf16), Bbc_ref[...],
                    preferred_element_type=jnp.float32) * (1.0 / (C * E))
    distf = distr / (dmean + 1e-6)
    attr3 = jnp.concatenate(
        [cosf, distf, jnp.ones((1, PL), jnp.float32)], axis=0).astype(bf16)

    inv_deg = 1.0 / float(C - 1)
    S2 = S2_ref[...]
    Sbig = Sbig_ref[...]

    for Wcat, WsT, bsT in ((Wcat_0_ref, WsT_0_ref, bsT_0_ref),
                           (Wcat_1_ref, WsT_1_ref, bsT_1_ref)):
        hrepT = jnp.dot(hT.astype(bf16), Rbig,
                        preferred_element_type=jnp.float32).astype(bf16)
        hrep_c = jnp.concatenate([hrepT] * (CHUNK // H), axis=0)
        msg_parts = []
        for c in range(NCHUNK):
            r0 = c * CHUNK
            pre = jnp.dot(Wcat[r0:r0 + CHUNK, :], attr3,
                          preferred_element_type=jnp.float32)
            M = jnp.tanh(pre).astype(bf16) * hrep_c
            msg_parts.append(
                jnp.dot(S2, M, preferred_element_type=jnp.float32))
        MSG = jnp.concatenate(msg_parts, axis=0)
        AGG = jnp.dot(MSG.astype(bf16), Sbig,
                      preferred_element_type=jnp.float32)
        hT = jnp.maximum(
            AGG * inv_deg
            + jnp.dot(WsT[...], hT, preferred_element_type=jnp.float32)
            + bsT[...], 0.0)

    yT_ref[0] = xT + jnp.dot(WoutT_ref[...], hT,
                             preferred_element_type=jnp.float32) + bout_ref[...]


def _routing_mats():
    q = np.arange(BPS * P)
    qb, qs, qd = q // P, (q % P) // C, q % C
    t = np.arange(BPS * C)
    tb, tn = t // C, t % C
    Rbig = (qb[None, :] == tb[:, None]) & (qs[None, :] == tn[:, None])
    Rdbig = (qb[None, :] == tb[:, None]) & (qd[None, :] == tn[:, None])
    Sbig = ((qb[:, None] == tb[None, :]) & (qd[:, None] == tn[None, :])
            & (qs[:, None] != tn[None, :]))
    Bsum = (qb[:, None] == tb[None, :])
    Bbc = (tb[:, None] == qb[None, :])
    k = np.arange(CHUNK)
    j = np.arange(CHUNK // H)
    S2 = (k[None, :] // H == j[:, None])
    bf = jnp.bfloat16
    return (jnp.asarray(Rbig, bf), jnp.asarray(Rdbig, bf),
            jnp.asarray(Sbig, bf), jnp.asarray(Bsum, bf),
            jnp.asarray(Bbc, bf), jnp.asarray(S2, bf))


def kernel(x, W_in, b_in, W_out, b_out,
           We_0, be_0, Ws_0, bs_0, We_1, be_1, Ws_1, bs_1):
    B = x.shape[0]
    f32 = jnp.float32
    G = B // BPS

    xT = (x.astype(f32).reshape(G, BPS, C, F)
          .transpose(0, 3, 1, 2).reshape(G, F, CL))
    WinT = W_in.T
    WoutT = W_out.T
    binT = b_in[:, None]
    boutT = b_out[:, None]

    def edge_cat(We, be):
        Wp = We.reshape(2, H, H).transpose(0, 2, 1).reshape(2, HH)
        bp = be.reshape(H, H).T.reshape(HH)
        return jnp.stack([Wp[0], Wp[1], bp], axis=1).astype(jnp.bfloat16)

    Wcat_0 = edge_cat(We_0, be_0)
    Wcat_1 = edge_cat(We_1, be_1)
    WsT_0, bsT_0 = Ws_0.T, bs_0[:, None]
    WsT_1, bsT_1 = Ws_1.T, bs_1[:, None]
    Rbig, Rdbig, Sbig, Bsum, Bbc, S2 = _routing_mats()

    full = lambda shape: pl.BlockSpec(shape, lambda b: (0,) * len(shape))
    grid_spec = pl.GridSpec(
        grid=(G,),
        in_specs=[
            pl.BlockSpec((1, F, CL), lambda b: (b, 0, 0)),
            full((H, F)), full((H, 1)), full((F, H)), full((F, 1)),
            full((CL, PL)), full((CL, PL)), full((PL, CL)),
            full((PL, CL)), full((CL, PL)), full((CHUNK // H, CHUNK)),
            full((HH, 3)), full((H, H)), full((H, 1)),
            full((HH, 3)), full((H, H)), full((H, 1)),
        ],
        out_specs=pl.BlockSpec((1, F, CL), lambda b: (b, 0, 0)),
    )
    yT = pl.pallas_call(
        _body,
        grid_spec=grid_spec,
        out_shape=jax.ShapeDtypeStruct((G, F, CL), f32),
        compiler_params=pltpu.CompilerParams(
            dimension_semantics=("parallel",)),
    )(xT, WinT, binT, WoutT, boutT,
      Rbig, Rdbig, Sbig, Bsum, Bbc, S2,
      Wcat_0, WsT_0, bsT_0, Wcat_1, WsT_1, bsT_1)
    return (yT.reshape(G, F, BPS, C).transpose(0, 2, 3, 1)
            .reshape(B, C, F))


# speedup vs baseline: 1.5834x; 1.1048x over previous
import numpy as np
import jax
import jax.numpy as jnp
from jax.experimental import pallas as pl
from jax.experimental.pallas import tpu as pltpu

C = 32
F = 256
H = 32
HH = H * H
P = C * C
E = C * (C - 1)
BPS = 4
PL = BPS * P
CL = BPS * C
CHUNK = 64
NCHUNK = HH // CHUNK


def _body(xT_ref, WinT_ref, bin_ref, WoutT_ref, bout_ref,
          Rbig_ref, Rdbig_ref, Sbig_ref, Bsum_ref, Bbc_ref, S2_ref,
          Wcat_0_ref, WsT_0_ref, bsT_0_ref,
          Wcat_1_ref, WsT_1_ref, bsT_1_ref,
          yT_ref):
    bf16 = jnp.bfloat16
    xT = xT_ref[0]

    hT = jnp.maximum(
        jnp.dot(WinT_ref[...], xT, preferred_element_type=jnp.float32)
        + bin_ref[...], 0.0)

    Rbig = Rbig_ref[...]
    hb0 = hT.astype(bf16)
    hsrcT = jnp.dot(hb0, Rbig, preferred_element_type=jnp.float32)
    hdstT = jnp.dot(hb0, Rdbig_ref[...], preferred_element_type=jnp.float32)
    numf = jnp.sum(hsrcT * hdstT, axis=0, keepdims=True)
    nsrc = jnp.maximum(
        jnp.sqrt(jnp.sum(hsrcT * hsrcT, axis=0, keepdims=True)), 1e-8)
    ndst = jnp.maximum(
        jnp.sqrt(jnp.sum(hdstT * hdstT, axis=0, keepdims=True)), 1e-8)
    cosf = numf / (nsrc * ndst)
    dvec = hdstT - hsrcT
    distr = jnp.sqrt(jnp.sum(dvec * dvec, axis=0, keepdims=True))
    bsums = jnp.dot(distr.astype(bf16), Bsum_ref[...],
                    preferred_element_type=jnp.float32)
    dmean = jnp.dot(bsums.astype(bf16), Bbc_ref[...],
                    preferred_element_type=jnp.float32) * (1.0 / (C * E))
    distf = distr / (dmean + 1e-6)
    attr3 = jnp.concatenate(
        [cosf, distf, jnp.ones((1, PL), jnp.float32)], axis=0).astype(bf16)

    inv_deg = 1.0 / float(C - 1)
    S2 = S2_ref[...]
    Sbig = Sbig_ref[...]

    for Wcat, WsT, bsT in ((Wcat_0_ref, WsT_0_ref, bsT_0_ref),
                           (Wcat_1_ref, WsT_1_ref, bsT_1_ref)):
        hrepT = jnp.dot(hT.astype(bf16), Rbig,
                        preferred_element_type=jnp.float32).astype(bf16)
        hrep_c = jnp.concatenate([hrepT] * (CHUNK // H), axis=0)
        msg_parts = []
        for c in range(NCHUNK):
            r0 = c * CHUNK
            pre = jnp.dot(Wcat[r0:r0 + CHUNK, :], attr3,
                          preferred_element_type=jnp.float32)
            M = jnp.tanh(pre).astype(bf16) * hrep_c
            msg_parts.append(
                jnp.dot(S2, M, preferred_element_type=jnp.float32))
        MSG = jnp.concatenate(msg_parts, axis=0)
        AGG = jnp.dot(MSG.astype(bf16), Sbig,
                      preferred_element_type=jnp.float32)
        hT = jnp.maximum(
            AGG * inv_deg
            + jnp.dot(WsT[...], hT, preferred_element_type=jnp.float32)
            + bsT[...], 0.0)

    yT_ref[0] = xT + jnp.dot(WoutT_ref[...], hT,
                             preferred_element_type=jnp.float32) + bout_ref[...]


def _routing_mats():
    q = np.arange(BPS * P)
    qb, qs, qd = q // P, (q % P) // C, q % C
    t = np.arange(BPS * C)
    tb, tn = t // C, t % C
    Rbig = (qb[None, :] == tb[:, None]) & (qs[None, :] == tn[:, None])
    Rdbig = (qb[None, :] == tb[:, None]) & (qd[None, :] == tn[:, None])
    Sbig = ((qb[:, None] == tb[None, :]) & (qd[:, None] == tn[None, :])
            & (qs[:, None] != tn[None, :]))
    Bsum = (qb[:, None] == tb[None, :])
    Bbc = (tb[:, None] == qb[None, :])
    k = np.arange(CHUNK)
    j = np.arange(CHUNK // H)
    S2 = (k[None, :] // H == j[:, None])
    bf = jnp.bfloat16
    return (jnp.asarray(Rbig, bf), jnp.asarray(Rdbig, bf),
            jnp.asarray(Sbig, bf), jnp.asarray(Bsum, bf),
            jnp.asarray(Bbc, bf), jnp.asarray(S2, bf))


def kernel(x, W_in, b_in, W_out, b_out,
           We_0, be_0, Ws_0, bs_0, We_1, be_1, Ws_1, bs_1):
    B = x.shape[0]
    f32 = jnp.float32
    G = B // BPS

    xT = (x.astype(f32).reshape(G, BPS, C, F)
          .transpose(0, 3, 1, 2).reshape(G, F, CL))
    WinT = W_in.T
    WoutT = W_out.T
    binT = b_in[:, None]
    boutT = b_out[:, None]

    def edge_cat(We, be):
        Wp = We.reshape(2, H, H).transpose(0, 2, 1).reshape(2, HH)
        bp = be.reshape(H, H).T.reshape(HH)
        return jnp.stack([Wp[0], Wp[1], bp], axis=1).astype(jnp.bfloat16)

    Wcat_0 = edge_cat(We_0, be_0)
    Wcat_1 = edge_cat(We_1, be_1)
    WsT_0, bsT_0 = Ws_0.T, bs_0[:, None]
    WsT_1, bsT_1 = Ws_1.T, bs_1[:, None]
    Rbig, Rdbig, Sbig, Bsum, Bbc, S2 = _routing_mats()

    full = lambda shape: pl.BlockSpec(shape, lambda b: (0,) * len(shape))
    grid_spec = pl.GridSpec(
        grid=(G,),
        in_specs=[
            pl.BlockSpec((1, F, CL), lambda b: (b, 0, 0)),
            full((H, F)), full((H, 1)), full((F, H)), full((F, 1)),
            full((CL, PL)), full((CL, PL)), full((PL, CL)),
            full((PL, CL)), full((CL, PL)), full((CHUNK // H, CHUNK)),
            full((HH, 3)), full((H, H)), full((H, 1)),
            full((HH, 3)), full((H, H)), full((H, 1)),
        ],
        out_specs=pl.BlockSpec((1, F, CL), lambda b: (b, 0, 0)),
    )
    yT = pl.pallas_call(
        _body,
        grid_spec=grid_spec,
        out_shape=jax.ShapeDtypeStruct((G, F, CL), f32),
        compiler_params=pltpu.CompilerParams(
            dimension_semantics=("parallel",)),
    )(xT, WinT, binT, WoutT, boutT,
      Rbig, Rdbig, Sbig, Bsum, Bbc, S2,
      Wcat_0, WsT_0, bsT_0, Wcat_1, WsT_1, bsT_1)
    return (yT.reshape(G, F, BPS, C).transpose(0, 2, 3, 1)
            .reshape(B, C, F))


# speedup vs baseline: 1.7311x; 1.0933x over previous
import numpy as np
import jax
import jax.numpy as jnp
from jax.experimental import pallas as pl
from jax.experimental.pallas import tpu as pltpu

C = 32
F = 256
H = 32
HH = H * H
P = C * C
E = C * (C - 1)
BPS = 4
PL = BPS * P
CL = BPS * C
CHUNK = 64
NCHUNK = HH // CHUNK


def _body(x_ref, Win_ref, bin_ref, Wout_ref, bout_ref,
          Rbig_ref, Rdbig_ref, Sbig_ref, Bsum_ref, Bbc_ref, S2_ref,
          Wcat_0_ref, WsT_0_ref, bsT_0_ref,
          Wcat_1_ref, WsT_1_ref, bsT_1_ref,
          y_ref):
    bf16 = jnp.bfloat16
    x2 = x_ref[0]

    h2 = jnp.maximum(
        jnp.dot(x2, Win_ref[...], preferred_element_type=jnp.float32)
        + bin_ref[...], 0.0)
    hT = h2.T

    Rbig = Rbig_ref[...]
    hb0 = hT.astype(bf16)
    hsrcT = jnp.dot(hb0, Rbig, preferred_element_type=jnp.float32)
    hdstT = jnp.dot(hb0, Rdbig_ref[...], preferred_element_type=jnp.float32)
    numf = jnp.sum(hsrcT * hdstT, axis=0, keepdims=True)
    nsrc = jnp.maximum(
        jnp.sqrt(jnp.sum(hsrcT * hsrcT, axis=0, keepdims=True)), 1e-8)
    ndst = jnp.maximum(
        jnp.sqrt(jnp.sum(hdstT * hdstT, axis=0, keepdims=True)), 1e-8)
    cosf = numf / (nsrc * ndst)
    dvec = hdstT - hsrcT
    distr = jnp.sqrt(jnp.sum(dvec * dvec, axis=0, keepdims=True))
    bsums = jnp.dot(distr.astype(bf16), Bsum_ref[...],
                    preferred_element_type=jnp.float32)
    dmean = jnp.dot(bsums.astype(bf16), Bbc_ref[...],
                    preferred_element_type=jnp.float32) * (1.0 / (C * E))
    distf = distr / (dmean + 1e-6)
    attr3 = jnp.concatenate(
        [cosf, distf, jnp.ones((1, PL), jnp.float32)], axis=0).astype(bf16)

    inv_deg = 1.0 / float(C - 1)
    S2 = S2_ref[...]
    Sbig = Sbig_ref[...]

    for Wcat, WsT, bsT in ((Wcat_0_ref, WsT_0_ref, bsT_0_ref),
                           (Wcat_1_ref, WsT_1_ref, bsT_1_ref)):
        hrepT = jnp.dot(hT.astype(bf16), Rbig,
                        preferred_element_type=jnp.float32).astype(bf16)
        hrep_c = jnp.concatenate([hrepT] * (CHUNK // H), axis=0)
        msg_parts = []
        for c in range(NCHUNK):
            r0 = c * CHUNK
            pre = jnp.dot(Wcat[r0:r0 + CHUNK, :], attr3,
                          preferred_element_type=jnp.float32)
            M = jnp.tanh(pre).astype(bf16) * hrep_c
            msg_parts.append(
                jnp.dot(S2, M, preferred_element_type=jnp.float32))
        MSG = jnp.concatenate(msg_parts, axis=0)
        AGG = jnp.dot(MSG.astype(bf16), Sbig,
                      preferred_element_type=jnp.float32)
        hT = jnp.maximum(
            AGG * inv_deg
            + jnp.dot(WsT[...], hT, preferred_element_type=jnp.float32)
            + bsT[...], 0.0)

    h1n = hT.T
    y_ref[0] = x2 + jnp.dot(h1n, Wout_ref[...],
                            preferred_element_type=jnp.float32) + bout_ref[...]


def _routing_mats():
    q = np.arange(BPS * P)
    qb, qs, qd = q // P, (q % P) // C, q % C
    t = np.arange(BPS * C)
    tb, tn = t // C, t % C
    Rbig = (qb[None, :] == tb[:, None]) & (qs[None, :] == tn[:, None])
    Rdbig = (qb[None, :] == tb[:, None]) & (qd[None, :] == tn[:, None])
    Sbig = ((qb[:, None] == tb[None, :]) & (qd[:, None] == tn[None, :])
            & (qs[:, None] != tn[None, :]))
    Bsum = (qb[:, None] == tb[None, :])
    Bbc = (tb[:, None] == qb[None, :])
    k = np.arange(CHUNK)
    j = np.arange(CHUNK // H)
    S2 = (k[None, :] // H == j[:, None])
    bf = jnp.bfloat16
    return (jnp.asarray(Rbig, bf), jnp.asarray(Rdbig, bf),
            jnp.asarray(Sbig, bf), jnp.asarray(Bsum, bf),
            jnp.asarray(Bbc, bf), jnp.asarray(S2, bf))


def kernel(x, W_in, b_in, W_out, b_out,
           We_0, be_0, Ws_0, bs_0, We_1, be_1, Ws_1, bs_1):
    B = x.shape[0]
    f32 = jnp.float32
    G = B // BPS

    xg = x.astype(f32).reshape(G, CL, F)
    bin_row = b_in[None, :]
    bout_row = b_out[None, :]

    def edge_cat(We, be):
        Wp = We.reshape(2, H, H).transpose(0, 2, 1).reshape(2, HH)
        bp = be.reshape(H, H).T.reshape(HH)
        return jnp.stack([Wp[0], Wp[1], bp], axis=1).astype(jnp.bfloat16)

    Wcat_0 = edge_cat(We_0, be_0)
    Wcat_1 = edge_cat(We_1, be_1)
    WsT_0, bsT_0 = Ws_0.T, bs_0[:, None]
    WsT_1, bsT_1 = Ws_1.T, bs_1[:, None]
    Rbig, Rdbig, Sbig, Bsum, Bbc, S2 = _routing_mats()

    full = lambda shape: pl.BlockSpec(shape, lambda b: (0,) * len(shape))
    grid_spec = pl.GridSpec(
        grid=(G,),
        in_specs=[
            pl.BlockSpec((1, CL, F), lambda b: (b, 0, 0)),
            full((F, H)), full((1, H)), full((H, F)), full((1, F)),
            full((CL, PL)), full((CL, PL)), full((PL, CL)),
            full((PL, CL)), full((CL, PL)), full((CHUNK // H, CHUNK)),
            full((HH, 3)), full((H, H)), full((H, 1)),
            full((HH, 3)), full((H, H)), full((H, 1)),
        ],
        out_specs=pl.BlockSpec((1, CL, F), lambda b: (b, 0, 0)),
    )
    yg = pl.pallas_call(
        _body,
        grid_spec=grid_spec,
        out_shape=jax.ShapeDtypeStruct((G, CL, F), f32),
        compiler_params=pltpu.CompilerParams(
            dimension_semantics=("parallel",)),
    )(xg, W_in, bin_row, W_out, bout_row,
      Rbig, Rdbig, Sbig, Bsum, Bbc, S2,
      Wcat_0, WsT_0, bsT_0, Wcat_1, WsT_1, bsT_1)
    return yg.reshape(B, C, F)


# speedup vs baseline: 2.0232x; 1.1687x over previous
import numpy as np
import jax
import jax.numpy as jnp
from jax.experimental import pallas as pl
from jax.experimental.pallas import tpu as pltpu

C = 32
F = 256
H = 32
HH = H * H
P = C * C
E = C * (C - 1)
BPS = 4
PL = BPS * P
CL = BPS * C
CHUNK = 128
NCHUNK = HH // CHUNK


def _body(x_ref, Win_ref, bin_ref, Wout_ref, bout_ref,
          Rbig_ref, Rdbig_ref, Sbig_ref, Bsum_ref, Bbc_ref, S2_ref,
          Wcat_0_ref, WsT_0_ref, bsT_0_ref,
          Wcat_1_ref, WsT_1_ref, bsT_1_ref,
          y_ref):
    bf16 = jnp.bfloat16
    x2 = x_ref[0]

    h2 = jnp.maximum(
        jnp.dot(x2, Win_ref[...], preferred_element_type=jnp.float32)
        + bin_ref[...], 0.0)
    hT = h2.T

    Rbig = Rbig_ref[...]
    hb0 = hT.astype(bf16)
    hsrcT = jnp.dot(hb0, Rbig, preferred_element_type=jnp.float32)
    hdstT = jnp.dot(hb0, Rdbig_ref[...], preferred_element_type=jnp.float32)
    numf = jnp.sum(hsrcT * hdstT, axis=0, keepdims=True)
    nsrc = jnp.maximum(
        jnp.sqrt(jnp.sum(hsrcT * hsrcT, axis=0, keepdims=True)), 1e-8)
    ndst = jnp.maximum(
        jnp.sqrt(jnp.sum(hdstT * hdstT, axis=0, keepdims=True)), 1e-8)
    cosf = numf / (nsrc * ndst)
    dvec = hdstT - hsrcT
    distr = jnp.sqrt(jnp.sum(dvec * dvec, axis=0, keepdims=True))
    bsums = jnp.dot(distr.astype(bf16), Bsum_ref[...],
                    preferred_element_type=jnp.float32)
    dmean = jnp.dot(bsums.astype(bf16), Bbc_ref[...],
                    preferred_element_type=jnp.float32) * (1.0 / (C * E))
    distf = distr / (dmean + 1e-6)
    attr3 = jnp.concatenate(
        [cosf, distf, jnp.ones((1, PL), jnp.float32)], axis=0).astype(bf16)

    inv_deg = 1.0 / float(C - 1)
    S2 = S2_ref[...]
    Sbig = Sbig_ref[...]

    for Wcat, WsT, bsT in ((Wcat_0_ref, WsT_0_ref, bsT_0_ref),
                           (Wcat_1_ref, WsT_1_ref, bsT_1_ref)):
        hrepT = jnp.dot(hT.astype(bf16), Rbig,
                        preferred_element_type=jnp.float32).astype(bf16)
        hrep_c = jnp.concatenate([hrepT] * (CHUNK // H), axis=0)
        msg_parts = []
        for c in range(NCHUNK):
            r0 = c * CHUNK
            pre = jnp.dot(Wcat[r0:r0 + CHUNK, :], attr3,
                          preferred_element_type=jnp.float32)
            M = jnp.tanh(pre).astype(bf16) * hrep_c
            msg_parts.append(
                jnp.dot(S2, M, preferred_element_type=jnp.float32))
        MSG = jnp.concatenate(msg_parts, axis=0)
        AGG = jnp.dot(MSG.astype(bf16), Sbig,
                      preferred_element_type=jnp.float32)
        hT = jnp.maximum(
            AGG * inv_deg
            + jnp.dot(WsT[...], hT, preferred_element_type=jnp.float32)
            + bsT[...], 0.0)

    h1n = hT.T
    y_ref[0] = x2 + jnp.dot(h1n, Wout_ref[...],
                            preferred_element_type=jnp.float32) + bout_ref[...]


def _routing_mats():
    q = np.arange(BPS * P)
    qb, qs, qd = q // P, (q % P) // C, q % C
    t = np.arange(BPS * C)
    tb, tn = t // C, t % C
    Rbig = (qb[None, :] == tb[:, None]) & (qs[None, :] == tn[:, None])
    Rdbig = (qb[None, :] == tb[:, None]) & (qd[None, :] == tn[:, None])
    Sbig = ((qb[:, None] == tb[None, :]) & (qd[:, None] == tn[None, :])
            & (qs[:, None] != tn[None, :]))
    Bsum = (qb[:, None] == tb[None, :])
    Bbc = (tb[:, None] == qb[None, :])
    k = np.arange(CHUNK)
    j = np.arange(CHUNK // H)
    S2 = (k[None, :] // H == j[:, None])
    bf = jnp.bfloat16
    return (jnp.asarray(Rbig, bf), jnp.asarray(Rdbig, bf),
            jnp.asarray(Sbig, bf), jnp.asarray(Bsum, bf),
            jnp.asarray(Bbc, bf), jnp.asarray(S2, bf))


def kernel(x, W_in, b_in, W_out, b_out,
           We_0, be_0, Ws_0, bs_0, We_1, be_1, Ws_1, bs_1):
    B = x.shape[0]
    f32 = jnp.float32
    G = B // BPS

    xg = x.astype(f32).reshape(G, CL, F)
    bin_row = b_in[None, :]
    bout_row = b_out[None, :]

    def edge_cat(We, be):
        Wp = We.reshape(2, H, H).transpose(0, 2, 1).reshape(2, HH)
        bp = be.reshape(H, H).T.reshape(HH)
        return jnp.stack([Wp[0], Wp[1], bp], axis=1).astype(jnp.bfloat16)

    Wcat_0 = edge_cat(We_0, be_0)
    Wcat_1 = edge_cat(We_1, be_1)
    WsT_0, bsT_0 = Ws_0.T, bs_0[:, None]
    WsT_1, bsT_1 = Ws_1.T, bs_1[:, None]
    Rbig, Rdbig, Sbig, Bsum, Bbc, S2 = _routing_mats()

    full = lambda shape: pl.BlockSpec(shape, lambda b: (0,) * len(shape))
    grid_spec = pl.GridSpec(
        grid=(G,),
        in_specs=[
            pl.BlockSpec((1, CL, F), lambda b: (b, 0, 0)),
            full((F, H)), full((1, H)), full((H, F)), full((1, F)),
            full((CL, PL)), full((CL, PL)), full((PL, CL)),
            full((PL, CL)), full((CL, PL)), full((CHUNK // H, CHUNK)),
            full((HH, 3)), full((H, H)), full((H, 1)),
            full((HH, 3)), full((H, H)), full((H, 1)),
        ],
        out_specs=pl.BlockSpec((1, CL, F), lambda b: (b, 0, 0)),
    )
    yg = pl.pallas_call(
        _body,
        grid_spec=grid_spec,
        out_shape=jax.ShapeDtypeStruct((G, CL, F), f32),
        compiler_params=pltpu.CompilerParams(
            dimension_semantics=("parallel",)),
    )(xg, W_in, bin_row, W_out, bout_row,
      Rbig, Rdbig, Sbig, Bsum, Bbc, S2,
      Wcat_0, WsT_0, bsT_0, Wcat_1, WsT_1, bsT_1)
    return yg.reshape(B, C, F)
